# Initial kernel scaffold; baseline (speedup 1.0000x reference)
import jax
import jax.numpy as jnp
from jax.experimental import pallas as pl


def kernel(hidden_states, gate_w, w1, w2, w3, s_w1, s_w2, s_w3):
    raise NotImplementedError("write your pallas kernel here")


# speedup vs baseline: 1.1345x; 1.1345x over previous
import jax
import jax.numpy as jnp
from jax.experimental import pallas as pl
from jax.experimental.pallas import tpu as pltpu

T = 128
D = 1024
E = 64
DFF = 512


def _nt_dot(a, b):
    return jax.lax.dot_general(
        a, b, (((1,), (1,)), ((), ())), preferred_element_type=jnp.float32
    )


def _moe_kernel(x_ref, gw_ref, w1_ref, w2_ref, w3_ref,
                sw1_ref, sw2_ref, sw3_ref,
                out_ref, gates_ref, acc_ref):
    e = pl.program_id(0)
    x = x_ref[...]

    @pl.when(e == 0)
    def _router():
        logits = _nt_dot(x, gw_ref[...])
        m = jnp.max(logits, axis=-1, keepdims=True)
        ex = jnp.exp(logits - m)
        probs = ex / jnp.sum(ex, axis=-1, keepdims=True)
        col = jax.lax.broadcasted_iota(jnp.int32, (T, E), 1)
        m1 = jnp.max(probs, axis=-1, keepdims=True)
        idx1 = jnp.min(jnp.where(probs == m1, col, E), axis=-1, keepdims=True)
        oh1 = col == idx1
        probs_m = jnp.where(oh1, -1.0, probs)
        m2 = jnp.max(probs_m, axis=-1, keepdims=True)
        idx2 = jnp.min(jnp.where(probs_m == m2, col, E), axis=-1, keepdims=True)
        oh2 = col == idx2
        denom = m1 + m2
        gates_ref[...] = (jnp.where(oh1, m1, 0.0) + jnp.where(oh2, m2, 0.0)) / denom

    @pl.when(e < E)
    def _expert():
        h1 = _nt_dot(x, w1_ref[0])
        h3 = _nt_dot(x, w3_ref[0])
        h = (h1 * jax.nn.sigmoid(h1)) * h3
        col = jax.lax.broadcasted_iota(jnp.int32, (T, E), 1)
        g = jnp.sum(jnp.where(col == e, gates_ref[...], 0.0),
                    axis=1, keepdims=True)
        h = h * g
        contrib = _nt_dot(h, w2_ref[0])
        @pl.when(e == 0)
        def _():
            acc_ref[...] = contrib
        @pl.when(e > 0)
        def _():
            acc_ref[...] += contrib

    @pl.when(e == E)
    def _shared():
        h1 = _nt_dot(x, sw1_ref[...])
        h3 = _nt_dot(x, sw3_ref[...])
        h = (h1 * jax.nn.sigmoid(h1)) * h3
        out_ref[...] = acc_ref[...] + _nt_dot(h, sw2_ref[...])


def kernel(hidden_states, gate_w, w1, w2, w3, s_w1, s_w2, s_w3):
    grid = (E + 1,)
    clamp = lambda e: (jnp.minimum(e, E - 1), 0, 0)
    full = lambda e: (0, 0)
    return pl.pallas_call(
        _moe_kernel,
        grid=grid,
        in_specs=[
            pl.BlockSpec((T, D), full),
            pl.BlockSpec((E, D), full),
            pl.BlockSpec((1, DFF, D), clamp),
            pl.BlockSpec((1, D, DFF), clamp),
            pl.BlockSpec((1, DFF, D), clamp),
            pl.BlockSpec((DFF, D), full),
            pl.BlockSpec((D, DFF), full),
            pl.BlockSpec((DFF, D), full),
        ],
        out_specs=pl.BlockSpec((T, D), full),
        out_shape=jax.ShapeDtypeStruct((T, D), jnp.float32),
        scratch_shapes=[
            pltpu.VMEM((T, E), jnp.float32),
            pltpu.VMEM((T, D), jnp.float32),
        ],
    )(hidden_states, gate_w, w1, w2, w3, s_w1, s_w2, s_w3)


# speedup vs baseline: 1.1382x; 1.0033x over previous
import jax
import jax.numpy as jnp
from jax.experimental import pallas as pl
from jax.experimental.pallas import tpu as pltpu

T = 128
D = 1024
E = 64
DFF = 512


def _nt_dot(a, b):
    return jax.lax.dot_general(
        a, b, (((1,), (1,)), ((), ())), preferred_element_type=jnp.float32
    )


def _moe_kernel(x_ref, gw_ref, w1_ref, w2_ref, w3_ref,
                sw1_ref, sw2_ref, sw3_ref,
                out_ref, gates_ref, acc_ref):
    e = pl.program_id(0)
    x = x_ref[...]

    @pl.when(e == 0)
    def _router():
        logits = _nt_dot(x, gw_ref[...])
        m = jnp.max(logits, axis=-1, keepdims=True)
        ex = jnp.exp(logits - m)
        probs = ex / jnp.sum(ex, axis=-1, keepdims=True)
        col = jax.lax.broadcasted_iota(jnp.int32, (T, E), 1)
        m1 = jnp.max(probs, axis=-1, keepdims=True)
        idx1 = jnp.min(jnp.where(probs == m1, col, E), axis=-1, keepdims=True)
        oh1 = col == idx1
        probs_m = jnp.where(oh1, -1.0, probs)
        m2 = jnp.max(probs_m, axis=-1, keepdims=True)
        idx2 = jnp.min(jnp.where(probs_m == m2, col, E), axis=-1, keepdims=True)
        oh2 = col == idx2
        denom = m1 + m2
        gates_ref[...] = (jnp.where(oh1, m1, 0.0) + jnp.where(oh2, m2, 0.0)) / denom

    @pl.when(e < E)
    def _expert():
        xb = x.astype(jnp.bfloat16)
        h1 = _nt_dot(xb, w1_ref[0].astype(jnp.bfloat16))
        h3 = _nt_dot(xb, w3_ref[0].astype(jnp.bfloat16))
        h = (h1 * jax.nn.sigmoid(h1)) * h3
        col = jax.lax.broadcasted_iota(jnp.int32, (T, E), 1)
        g = jnp.sum(jnp.where(col == e, gates_ref[...], 0.0),
                    axis=1, keepdims=True)
        h = h * g
        contrib = _nt_dot(h.astype(jnp.bfloat16),
                          w2_ref[0].astype(jnp.bfloat16))
        @pl.when(e == 0)
        def _():
            acc_ref[...] = contrib
        @pl.when(e > 0)
        def _():
            acc_ref[...] += contrib

    @pl.when(e == E)
    def _shared():
        h1 = _nt_dot(x, sw1_ref[...])
        h3 = _nt_dot(x, sw3_ref[...])
        h = (h1 * jax.nn.sigmoid(h1)) * h3
        out_ref[...] = acc_ref[...] + _nt_dot(h, sw2_ref[...])


def kernel(hidden_states, gate_w, w1, w2, w3, s_w1, s_w2, s_w3):
    grid = (E + 1,)
    clamp = lambda e: (jnp.minimum(e, E - 1), 0, 0)
    full = lambda e: (0, 0)
    return pl.pallas_call(
        _moe_kernel,
        grid=grid,
        in_specs=[
            pl.BlockSpec((T, D), full),
            pl.BlockSpec((E, D), full),
            pl.BlockSpec((1, DFF, D), clamp),
            pl.BlockSpec((1, D, DFF), clamp),
            pl.BlockSpec((1, DFF, D), clamp),
            pl.BlockSpec((DFF, D), full),
            pl.BlockSpec((D, DFF), full),
            pl.BlockSpec((DFF, D), full),
        ],
        out_specs=pl.BlockSpec((T, D), full),
        out_shape=jax.ShapeDtypeStruct((T, D), jnp.float32),
        scratch_shapes=[
            pltpu.VMEM((T, E), jnp.float32),
            pltpu.VMEM((T, D), jnp.float32),
        ],
    )(hidden_states, gate_w, w1, w2, w3, s_w1, s_w2, s_w3)


# speedup vs baseline: 1.2841x; 1.1281x over previous
import jax
import jax.numpy as jnp
from jax.experimental import pallas as pl
from jax.experimental.pallas import tpu as pltpu

T = 128
D = 1024
E = 64
DFF = 512
EPB = 2
NEB = E // EPB


def _nt_dot(a, b):
    return jax.lax.dot_general(
        a, b, (((1,), (1,)), ((), ())), preferred_element_type=jnp.float32
    )


def _moe_kernel(x_ref, gw_ref, w1_ref, w2_ref, w3_ref,
                sw1_ref, sw2_ref, sw3_ref,
                out_ref, gates_ref, acc_ref, xb_ref):
    i = pl.program_id(0)
    x = x_ref[...]

    @pl.when(i == 0)
    def _router():
        xb_ref[...] = x.astype(jnp.bfloat16)
        logits = _nt_dot(x, gw_ref[...])
        m = jnp.max(logits, axis=-1, keepdims=True)
        ex = jnp.exp(logits - m)
        probs = ex / jnp.sum(ex, axis=-1, keepdims=True)
        col = jax.lax.broadcasted_iota(jnp.int32, (T, E), 1)
        m1 = jnp.max(probs, axis=-1, keepdims=True)
        idx1 = jnp.min(jnp.where(probs == m1, col, E), axis=-1, keepdims=True)
        oh1 = col == idx1
        probs_m = jnp.where(oh1, -1.0, probs)
        m2 = jnp.max(probs_m, axis=-1, keepdims=True)
        idx2 = jnp.min(jnp.where(probs_m == m2, col, E), axis=-1, keepdims=True)
        oh2 = col == idx2
        denom = m1 + m2
        gates_ref[...] = (jnp.where(oh1, m1, 0.0) + jnp.where(oh2, m2, 0.0)) / denom

    @pl.when(i < NEB)
    def _experts():
        xb = xb_ref[...]
        gates = gates_ref[...]
        col = jax.lax.broadcasted_iota(jnp.int32, (T, E), 1)
        total = jnp.zeros((T, D), jnp.float32)
        for j in range(EPB):
            e = i * EPB + j
            h1 = _nt_dot(xb, w1_ref[j].astype(jnp.bfloat16))
            h3 = _nt_dot(xb, w3_ref[j].astype(jnp.bfloat16))
            h = (h1 * jax.nn.sigmoid(h1)) * h3
            g = jnp.sum(jnp.where(col == e, gates, 0.0),
                        axis=1, keepdims=True)
            h = h * g
            total = total + _nt_dot(h.astype(jnp.bfloat16),
                                    w2_ref[j].astype(jnp.bfloat16))
        @pl.when(i == 0)
        def _():
            acc_ref[...] = total
        @pl.when(i > 0)
        def _():
            acc_ref[...] += total

    @pl.when(i == NEB)
    def _shared():
        h1 = _nt_dot(x, sw1_ref[...])
        h3 = _nt_dot(x, sw3_ref[...])
        h = (h1 * jax.nn.sigmoid(h1)) * h3
        out_ref[...] = acc_ref[...] + _nt_dot(h, sw2_ref[...])


def kernel(hidden_states, gate_w, w1, w2, w3, s_w1, s_w2, s_w3):
    grid = (NEB + 1,)
    clamp = lambda i: (jnp.minimum(i, NEB - 1), 0, 0)
    full = lambda i: (0, 0)
    return pl.pallas_call(
        _moe_kernel,
        grid=grid,
        in_specs=[
            pl.BlockSpec((T, D), full),
            pl.BlockSpec((E, D), full),
            pl.BlockSpec((EPB, DFF, D), clamp),
            pl.BlockSpec((EPB, D, DFF), clamp),
            pl.BlockSpec((EPB, DFF, D), clamp),
            pl.BlockSpec((DFF, D), full),
            pl.BlockSpec((D, DFF), full),
            pl.BlockSpec((DFF, D), full),
        ],
        out_specs=pl.BlockSpec((T, D), full),
        out_shape=jax.ShapeDtypeStruct((T, D), jnp.float32),
        scratch_shapes=[
            pltpu.VMEM((T, E), jnp.float32),
            pltpu.VMEM((T, D), jnp.float32),
            pltpu.VMEM((T, D), jnp.bfloat16),
        ],
    )(hidden_states, gate_w, w1, w2, w3, s_w1, s_w2, s_w3)
